# baseline (speedup 1.0000x reference)
import functools

import jax
import jax.numpy as jnp
from jax import lax
from jax.experimental import pallas as pl
from jax.experimental.pallas import tpu as pltpu, tpu_sc as plsc

_CHUNKS = 9


def _argmin_body(x_ref, cb_ref, idx_ref):
    cb = cb_ref[...]
    norms = jax.lax.dot_general(
        jnp.full((8, cb.shape[1]), 0.5, jnp.float32), cb * cb,
        (((1,), (1,)), ((), ())),
        precision=jax.lax.Precision.HIGHEST,
        preferred_element_type=jnp.float32)[0:1, :]
    R = x_ref.shape[0]
    C = R // _CHUNKS
    for c in range(_CHUNKS):
        xt = x_ref[c * C:(c + 1) * C, :]
        s = jax.lax.dot_general(
            xt, cb, (((1,), (1,)), ((), ())),
            preferred_element_type=jnp.float32)
        s = s - norms
        idx_ref[c * C:(c + 1) * C] = jnp.argmax(s, axis=1)


_SC_CHUNKS = 2


def _sc_gather_body(nc, b_per_w, cb_hbm, idx_hbm, out_hbm, idx_v, rows_v,
                    gsem0, gsem1, wsem):
    wid = lax.axis_index("s") * nc + lax.axis_index("c")
    base = wid * b_per_w
    cb = b_per_w // _SC_CHUNKS
    pltpu.sync_copy(idx_hbm.at[pl.ds(base, b_per_w)], idx_v)
    gsems = [gsem0, gsem1]
    gathers = [
        pltpu.async_copy(cb_hbm.at[idx_v.at[pl.ds(c * cb, cb)]],
                         rows_v.at[pl.ds(c * cb, cb)], gsems[c])
        for c in range(_SC_CHUNKS)
    ]
    writes = []
    for c in range(_SC_CHUNKS):
        gathers[c].wait()
        writes.append(
            pltpu.async_copy(rows_v.at[pl.ds(c * cb, cb)],
                             out_hbm.at[pl.ds(base + c * cb, cb)], wsem))
    for w in writes:
        w.wait()


_SPLITS = 2


def kernel(x, codebook):
    B, TOK, D = x.shape
    K, _ = codebook.shape
    N = B * TOK
    flat = x.reshape(N, D)
    NH = N // _SPLITS

    info = plsc.get_sparse_core_info()
    nc, ns = info.num_cores, info.num_subcores
    nw = nc * ns
    b_per_w = NH // nw
    DP = 128
    cbp = jnp.pad(codebook, ((0, 0), (0, DP - D)))
    mesh = plsc.VectorSubcoreMesh(core_axis_name="c", subcore_axis_name="s")
    gather = functools.partial(
        pl.kernel,
        mesh=mesh,
        out_type=jax.ShapeDtypeStruct((NH, DP), jnp.float32),
        scratch_types=[
            pltpu.VMEM((b_per_w,), jnp.int32),
            pltpu.VMEM((b_per_w, DP), jnp.float32),
            pltpu.SemaphoreType.DMA,
            pltpu.SemaphoreType.DMA,
            pltpu.SemaphoreType.DMA,
        ],
    )(functools.partial(_sc_gather_body, nc, b_per_w))

    argmax = lambda xs: pl.pallas_call(
        _argmin_body,
        grid=(1,),
        in_specs=[
            pl.BlockSpec((NH, D), lambda i: (0, 0)),
            pl.BlockSpec((K, D), lambda i: (0, 0)),
        ],
        out_specs=pl.BlockSpec((NH,), lambda i: (0,)),
        out_shape=jax.ShapeDtypeStruct((NH,), jnp.int32),
    )(xs, codebook)

    qs = [gather(cbp, argmax(flat[h * NH:(h + 1) * NH])) for h in range(_SPLITS)]
    q = jnp.concatenate([qh[:, :D] for qh in qs], axis=0)
    return q.reshape(x.shape)


# speedup vs baseline: 1.1054x; 1.1054x over previous
import functools

import jax
import jax.numpy as jnp
from jax import lax
from jax.experimental import pallas as pl
from jax.experimental.pallas import tpu as pltpu, tpu_sc as plsc

_CHUNKS = 9


def _argmin_body(x_ref, cb_ref, idx_ref):
    cb = cb_ref[...]
    norms = jax.lax.dot_general(
        jnp.full((8, cb.shape[1]), 0.5, jnp.float32), cb * cb,
        (((1,), (1,)), ((), ())),
        precision=jax.lax.Precision.HIGHEST,
        preferred_element_type=jnp.float32)[0:1, :]
    R = x_ref.shape[0]
    C = R // _CHUNKS
    for c in range(_CHUNKS):
        xt = x_ref[c * C:(c + 1) * C, :]
        s = jax.lax.dot_general(
            xt, cb, (((1,), (1,)), ((), ())),
            preferred_element_type=jnp.float32)
        s = s - norms
        idx_ref[c * C:(c + 1) * C] = jnp.argmax(s, axis=1)


def _sc_gather_body(nc, b_per_w, cb_hbm, idx_hbm, out_hbm, idx_v, rows_v, sem):
    wid = lax.axis_index("s") * nc + lax.axis_index("c")
    base = wid * b_per_w
    pltpu.sync_copy(idx_hbm.at[pl.ds(base, b_per_w)], idx_v)
    pltpu.async_copy(cb_hbm.at[idx_v], rows_v, sem).wait()
    pltpu.sync_copy(rows_v, out_hbm.at[pl.ds(base, b_per_w)])


def kernel(x, codebook):
    B, TOK, D = x.shape
    K, _ = codebook.shape
    N = B * TOK
    flat = x.reshape(N, D)
    idx = pl.pallas_call(
        _argmin_body,
        grid=(1,),
        in_specs=[
            pl.BlockSpec((N, D), lambda i: (0, 0)),
            pl.BlockSpec((K, D), lambda i: (0, 0)),
        ],
        out_specs=pl.BlockSpec((N,), lambda i: (0,)),
        out_shape=jax.ShapeDtypeStruct((N,), jnp.int32),
    )(flat, codebook)

    info = plsc.get_sparse_core_info()
    nc, ns = info.num_cores, info.num_subcores
    nw = nc * ns
    b_per_w = N // nw
    DP = 128
    cbp = jnp.pad(codebook, ((0, 0), (0, DP - D)))
    mesh = plsc.VectorSubcoreMesh(core_axis_name="c", subcore_axis_name="s")
    gather = functools.partial(
        pl.kernel,
        mesh=mesh,
        out_type=jax.ShapeDtypeStruct((N, DP), jnp.float32),
        scratch_types=[
            pltpu.VMEM((b_per_w,), jnp.int32),
            pltpu.VMEM((b_per_w, DP), jnp.float32),
            pltpu.SemaphoreType.DMA,
        ],
    )(functools.partial(_sc_gather_body, nc, b_per_w))
    q = gather(cbp, idx)
    return q[:, :D].reshape(x.shape)


# speedup vs baseline: 1.1281x; 1.0205x over previous
import functools

import jax
import jax.numpy as jnp
from jax import lax
from jax.experimental import pallas as pl
from jax.experimental.pallas import tpu as pltpu, tpu_sc as plsc

_CHUNKS = 12


def _argmin_body(x_ref, cb_ref, idx_ref):
    cb = cb_ref[...]
    norms = jax.lax.dot_general(
        jnp.full((8, cb.shape[1]), 0.5, jnp.float32), cb * cb,
        (((1,), (1,)), ((), ())),
        precision=jax.lax.Precision.HIGHEST,
        preferred_element_type=jnp.float32)[0:1, :]
    R = x_ref.shape[0]
    C = R // _CHUNKS
    for c in range(_CHUNKS):
        xt = x_ref[c * C:(c + 1) * C, :]
        s = jax.lax.dot_general(
            xt, cb, (((1,), (1,)), ((), ())),
            preferred_element_type=jnp.float32)
        s = s - norms
        idx_ref[c * C:(c + 1) * C] = jnp.argmax(s, axis=1)


def _sc_gather_body(nc, b_per_w, cb_hbm, idx_hbm, out_hbm, idx_v, rows_v, sem):
    wid = lax.axis_index("s") * nc + lax.axis_index("c")
    base = wid * b_per_w
    pltpu.sync_copy(idx_hbm.at[pl.ds(base, b_per_w)], idx_v)
    pltpu.async_copy(cb_hbm.at[idx_v], rows_v, sem).wait()
    pltpu.sync_copy(rows_v, out_hbm.at[pl.ds(base, b_per_w)])


def kernel(x, codebook):
    B, TOK, D = x.shape
    K, _ = codebook.shape
    N = B * TOK
    flat = x.reshape(N, D)
    idx = pl.pallas_call(
        _argmin_body,
        grid=(1,),
        in_specs=[
            pl.BlockSpec((N, D), lambda i: (0, 0)),
            pl.BlockSpec((K, D), lambda i: (0, 0)),
        ],
        out_specs=pl.BlockSpec((N,), lambda i: (0,)),
        out_shape=jax.ShapeDtypeStruct((N,), jnp.int32),
    )(flat, codebook)

    info = plsc.get_sparse_core_info()
    nc, ns = info.num_cores, info.num_subcores
    nw = nc * ns
    b_per_w = N // nw
    DP = 128
    cbp = jnp.pad(codebook, ((0, 0), (0, DP - D)))
    mesh = plsc.VectorSubcoreMesh(core_axis_name="c", subcore_axis_name="s")
    gather = functools.partial(
        pl.kernel,
        mesh=mesh,
        out_type=jax.ShapeDtypeStruct((N, DP), jnp.float32),
        scratch_types=[
            pltpu.VMEM((b_per_w,), jnp.int32),
            pltpu.VMEM((b_per_w, DP), jnp.float32),
            pltpu.SemaphoreType.DMA,
        ],
    )(functools.partial(_sc_gather_body, nc, b_per_w))
    q = gather(cbp, idx)
    return q[:, :D].reshape(x.shape)
